# Initial kernel scaffold; baseline (speedup 1.0000x reference)
import jax
import jax.numpy as jnp
from jax.experimental import pallas as pl


def kernel(x, edge_index, W1, b1, W2, b2, W3, b3, Wo, bo):
    raise NotImplementedError("write your pallas kernel here")


# speedup vs baseline: 2.4864x; 2.4864x over previous
import jax
import jax.numpy as jnp
from jax import lax
from jax.experimental import pallas as pl
from jax.experimental.pallas import tpu as pltpu

N = 10000
E = 320000
H = 128


def _mm_body(x_ref, w_ref, o_ref):
    o_ref[...] = jnp.dot(x_ref[...], w_ref[...], preferred_element_type=jnp.float32)


def _mm(x, w):
    return pl.pallas_call(
        _mm_body,
        out_shape=jax.ShapeDtypeStruct((x.shape[0], w.shape[1]), jnp.float32),
    )(x, w)


def kernel(x, edge_index, W1, b1, W2, b2, W3, b3, Wo, bo):
    src = edge_index[0]
    dst = edge_index[1]
    deg = 1.0 + jnp.zeros((N,), jnp.float32).at[dst].add(1.0)
    dinv = deg ** -0.5

    h = x
    for W, b in ((W1, b1), (W2, b2), (W3, b3)):
        xw = _mm(h, W)
        y = dinv[:, None] * xw
        agg = y + jnp.zeros_like(y).at[dst].add(y[src])
        h = jax.nn.relu(dinv[:, None] * agg + b)
    z = _mm(h, Wo) + bo
    return (h, z)


# speedup vs baseline: 2.6665x; 1.0724x over previous
import jax
import jax.numpy as jnp
from jax import lax
from jax.experimental import pallas as pl

N = 10000
E = 320000
H = 128


def _mm_body(x_ref, w_ref, o_ref):
    o_ref[...] = jnp.dot(x_ref[...], w_ref[...],
                         preferred_element_type=jnp.float32)


def _mm(x, w):
    return pl.pallas_call(
        _mm_body,
        out_shape=jax.ShapeDtypeStruct((x.shape[0], w.shape[1]), jnp.float32),
    )(x, w)


def _dinv_scale_body(xw_ref, deg_ref, y_ref, dinv_ref):
    dinv = lax.rsqrt(1.0 + deg_ref[...])
    dinv_ref[...] = dinv
    y_ref[...] = xw_ref[...] * dinv


def _dinv_scale(xw, deg):
    return pl.pallas_call(
        _dinv_scale_body,
        out_shape=(
            jax.ShapeDtypeStruct((N, H), jnp.float32),
            jax.ShapeDtypeStruct((N, 1), jnp.float32),
        ),
    )(xw, deg)


def _layer_body(y_ref, ap_ref, dinv_ref, b_ref, w_ref, yn_ref):
    agg = y_ref[...] + ap_ref[...]
    h = jnp.maximum(agg * dinv_ref[...] + b_ref[...], 0.0)
    yn_ref[...] = (jnp.dot(h, w_ref[...], preferred_element_type=jnp.float32)
                   * dinv_ref[...])


def _layer(y, ap, dinv, b, w):
    return pl.pallas_call(
        _layer_body,
        out_shape=jax.ShapeDtypeStruct((N, H), jnp.float32),
    )(y, ap, dinv, b, w)


def _final_body(y_ref, ap_ref, dinv_ref, b_ref, wo_ref, bo_ref, h_ref, z_ref):
    agg = y_ref[...] + ap_ref[...]
    h = jnp.maximum(agg * dinv_ref[...] + b_ref[...], 0.0)
    h_ref[...] = h
    z_ref[...] = (jnp.dot(h, wo_ref[...], preferred_element_type=jnp.float32)
                  + bo_ref[...])


def _final(y, ap, dinv, b, wo, bo):
    return pl.pallas_call(
        _final_body,
        out_shape=(
            jax.ShapeDtypeStruct((N, H), jnp.float32),
            jax.ShapeDtypeStruct((N, wo.shape[1]), jnp.float32),
        ),
    )(y, ap, dinv, b, wo, bo)


def kernel(x, edge_index, W1, b1, W2, b2, W3, b3, Wo, bo):
    src = edge_index[0]
    dst = edge_index[1]
    deg = jnp.zeros((N, 1), jnp.float32).at[dst, 0].add(1.0)

    xw1 = _mm(x, W1)
    y1, dinv = _dinv_scale(xw1, deg)

    def _scatter(y):
        return jnp.zeros_like(y).at[dst].add(y[src])

    y2 = _layer(y1, _scatter(y1), dinv, b1, W2)
    y3 = _layer(y2, _scatter(y2), dinv, b2, W3)
    h, z = _final(y3, _scatter(y3), dinv, b3, Wo, bo)
    return (h, z)
